# baseline (speedup 1.0000x reference)
import jax
import jax.numpy as jnp
from jax import lax
from jax.experimental import pallas as pl
from jax.experimental.pallas import tpu as pltpu
from jax.experimental.pallas import tpu_sc as plsc

_N = 10000
_E = 320000
_D = 128
_NCLASS = 10
_NGRAPH = 64

_NC = 2
_NS = 16
_N_PAD = 10240
_RPT = _N_PAD // _NS
_SCHUNK = 64
_SGROUP = 32
_NGROUPS = 10
_E_PAD = _NS * _NGROUPS * _SGROUP * _SCHUNK
_NBUF = 2
_SLAB = 32
_NSLAB = _RPT // _SLAB
_RB = 256
_GRID = _N_PAD // _RB

_f32 = jnp.float32
_i32 = jnp.int32


def _sc_mesh():
    return plsc.VectorSubcoreMesh(
        core_axis_name="c", subcore_axis_name="s",
        num_cores=_NC, num_subcores=_NS)


def _quake_rsqrt(d):
    i = lax.bitcast_convert_type(d, _i32)
    i = jnp.full((16,), 0x5F3759DF, _i32) - lax.shift_right_logical(
        i, jnp.full((16,), 1, _i32))
    y = lax.bitcast_convert_type(i, _f32)
    half = d * 0.5
    y = y * (1.5 - half * y * y)
    y = y * (1.5 - half * y * y)
    y = y * (1.5 - half * y * y)
    return y


def _edge_pipeline(table, h_hbm, src_v, dst_v, rows_v, gsems, ssems):
    gd = [None] * _SGROUP
    sd = [None] * _SGROUP

    def issue_gather(k):
        b = k % _NBUF
        gd[k] = pltpu.async_copy(
            h_hbm.at[src_v.at[k]], rows_v.at[b], gsems[b])

    issue_gather(0)
    for k in range(_SGROUP):
        b = k % _NBUF
        if k + 1 < _SGROUP:
            if k + 1 >= _NBUF:
                sd[k + 1 - _NBUF].wait()
            issue_gather(k + 1)
        gd[k].wait()
        sd[k] = pltpu.async_copy(
            rows_v.at[b], table.at[dst_v.at[k]], ssems[b], add=True)
    for k in range(_SGROUP - _NBUF, _SGROUP):
        sd[k].wait()


def _sc_a_body(x_hbm, src_hbm, dst_hbm, ones_hbm, zeros_hbm,
               xt0_hbm, xt1_hbm, inv0_hbm, inv1_hbm, u_hbm,
               table, src_v, dst_v, ones_v, rows_v, xbuf, degbuf, invbuf,
               *sems):
    c = lax.axis_index("c")
    s = lax.axis_index("s")
    row0 = s * _RPT
    gsems = sems[:_NBUF]
    ssems = sems[_NBUF:]

    pltpu.sync_copy(zeros_hbm, table.at[pl.ds(row0, _RPT)])
    pltpu.sync_copy(ones_hbm, ones_v)
    plsc.subcore_barrier()

    def deg_group(g, carry):
        pltpu.sync_copy(dst_hbm.at[c, s, g], dst_v)

        def body(j, carry2):
            pltpu.sync_copy(ones_v, table.at[dst_v.at[j]], add=True)
            return carry2

        lax.fori_loop(0, _SGROUP, body, 0)
        return carry

    lax.fori_loop(0, _NGROUPS, deg_group, 0)
    plsc.subcore_barrier()

    def run_scale(xt_hbm, inv_hbm):
        def slab(t, carry):
            r0 = row0 + t * _SLAB
            pltpu.sync_copy(x_hbm.at[pl.ds(r0, _SLAB)], xbuf)
            pltpu.sync_copy(table.at[pl.ds(r0, _SLAB)], degbuf)

            def row(r, carry2):
                d = degbuf[r, pl.ds(0, 16)] + 1.0
                inv = _quake_rsqrt(d)
                invbuf[r, pl.ds(0, 16)] = inv
                for q in range(_D // 16):
                    xbuf[r, pl.ds(q * 16, 16)] = (
                        xbuf[r, pl.ds(q * 16, 16)] * inv)
                return carry2

            lax.fori_loop(0, _SLAB, row, 0)
            pltpu.sync_copy(xbuf, xt_hbm.at[pl.ds(r0, _SLAB)])
            pltpu.sync_copy(invbuf, inv_hbm.at[pl.ds(r0, _SLAB)])
            return carry

        lax.fori_loop(0, _NSLAB, slab, 0)

    @pl.when(c == 0)
    def _():
        run_scale(xt0_hbm, inv0_hbm)

    @pl.when(c == 1)
    def _():
        run_scale(xt1_hbm, inv1_hbm)

    plsc.subcore_barrier()

    pltpu.sync_copy(zeros_hbm, table.at[pl.ds(row0, _RPT)])
    plsc.subcore_barrier()

    def run_edges(xt_hbm):
        def group(g, carry):
            pltpu.sync_copy(src_hbm.at[c, s, g], src_v)
            pltpu.sync_copy(dst_hbm.at[c, s, g], dst_v)
            _edge_pipeline(table, xt_hbm, src_v, dst_v, rows_v, gsems, ssems)
            return carry

        lax.fori_loop(0, _NGROUPS, group, 0)

    @pl.when(c == 0)
    def _():
        run_edges(xt0_hbm)

    @pl.when(c == 1)
    def _():
        run_edges(xt1_hbm)

    plsc.subcore_barrier()
    pltpu.sync_copy(table.at[pl.ds(row0, _RPT)],
                    u_hbm.at[c, pl.ds(row0, _RPT)])


def _sc_a(x_pad, src_all, dst_all, ones_rows, zrows):
    k = pl.kernel(
        _sc_a_body,
        out_type=[
            jax.ShapeDtypeStruct((_N_PAD, _D), _f32),
            jax.ShapeDtypeStruct((_N_PAD, _D), _f32),
            jax.ShapeDtypeStruct((_N_PAD, 16), _f32),
            jax.ShapeDtypeStruct((_N_PAD, 16), _f32),
            jax.ShapeDtypeStruct((_NC, _N_PAD, _D), _f32),
        ],
        mesh=_sc_mesh(),
        scratch_types=[
            pltpu.VMEM_SHARED((_N_PAD, _D), _f32),
            pltpu.VMEM((_SGROUP, _SCHUNK), _i32),
            pltpu.VMEM((_SGROUP, _SCHUNK), _i32),
            pltpu.VMEM((_SCHUNK, _D), _f32),
            pltpu.VMEM((_NBUF, _SCHUNK, _D), _f32),
            pltpu.VMEM((_SLAB, _D), _f32),
            pltpu.VMEM((_SLAB, _D), _f32),
            pltpu.VMEM((_SLAB, 16), _f32),
        ] + [pltpu.SemaphoreType.DMA] * (2 * _NBUF),
    )
    return k(x_pad, src_all, dst_all, ones_rows, zrows)


def _sc_b_body(h0_hbm, h1_hbm, src_hbm, dst_hbm, zeros_hbm, out_hbm,
               table, src_v, dst_v, rows_v, *sems):
    c = lax.axis_index("c")
    s = lax.axis_index("s")
    row0 = s * _RPT
    gsems = sems[:_NBUF]
    ssems = sems[_NBUF:]
    pltpu.sync_copy(zeros_hbm, table.at[pl.ds(row0, _RPT)])
    plsc.subcore_barrier()

    def run_edges(h_hbm):
        def group(g, carry):
            pltpu.sync_copy(src_hbm.at[c, s, g], src_v)
            pltpu.sync_copy(dst_hbm.at[c, s, g], dst_v)
            _edge_pipeline(table, h_hbm, src_v, dst_v, rows_v, gsems, ssems)
            return carry

        lax.fori_loop(0, _NGROUPS, group, 0)

    @pl.when(c == 0)
    def _():
        run_edges(h0_hbm)

    @pl.when(c == 1)
    def _():
        run_edges(h1_hbm)

    plsc.subcore_barrier()
    pltpu.sync_copy(table.at[pl.ds(row0, _RPT)],
                    out_hbm.at[c, pl.ds(row0, _RPT)])


def _sc_b(h0, h1, src_all, dst_all, zrows):
    k = pl.kernel(
        _sc_b_body,
        out_type=jax.ShapeDtypeStruct((_NC, _N_PAD, _D), _f32),
        mesh=_sc_mesh(),
        scratch_types=[
            pltpu.VMEM_SHARED((_N_PAD, _D), _f32),
            pltpu.VMEM((_SGROUP, _SCHUNK), _i32),
            pltpu.VMEM((_SGROUP, _SCHUNK), _i32),
            pltpu.VMEM((_NBUF, _SCHUNK, _D), _f32),
        ] + [pltpu.SemaphoreType.DMA] * (2 * _NBUF),
    )
    return k(h0, h1, src_all, dst_all, zrows)


def _tc1_body(u_ref, xt0_ref, xt1_ref, inv0_ref, inv1_ref, w0_ref, w1_ref,
              y0_ref, y1_ref):
    inv0 = inv0_ref[...][:, 0:1]
    inv1 = inv1_ref[...][:, 0:1]
    v0 = (u_ref[0] + xt0_ref[...]) * inv0
    v1 = (u_ref[1] + xt1_ref[...]) * inv1
    a0 = jnp.maximum(jnp.dot(v0, w0_ref[...], preferred_element_type=_f32), 0.0)
    a1 = jnp.maximum(jnp.dot(v1, w1_ref[...], preferred_element_type=_f32), 0.0)
    y0_ref[...] = a0 * inv0
    y1_ref[...] = a1 * inv1


def _tc1(u, xt0, xt1, inv0, inv1, W00, W10):
    return pl.pallas_call(
        _tc1_body,
        grid=(_GRID,),
        in_specs=[
            pl.BlockSpec((_NC, _RB, _D), lambda i: (0, i, 0)),
            pl.BlockSpec((_RB, _D), lambda i: (i, 0)),
            pl.BlockSpec((_RB, _D), lambda i: (i, 0)),
            pl.BlockSpec((_RB, 16), lambda i: (i, 0)),
            pl.BlockSpec((_RB, 16), lambda i: (i, 0)),
            pl.BlockSpec((_D, _D), lambda i: (0, 0)),
            pl.BlockSpec((_D, _D), lambda i: (0, 0)),
        ],
        out_specs=[
            pl.BlockSpec((_RB, _D), lambda i: (i, 0)),
            pl.BlockSpec((_RB, _D), lambda i: (i, 0)),
        ],
        out_shape=[
            jax.ShapeDtypeStruct((_N_PAD, _D), _f32),
            jax.ShapeDtypeStruct((_N_PAD, _D), _f32),
        ],
    )(u, xt0, xt1, inv0, inv1, W00, W10)


def _tc2_body(u2_ref, y0_ref, y1_ref, inv0_ref, inv1_ref, w0_ref, w1_ref,
              wout_ref, bout_ref, batch_ref, out_ref, sums, cnts):
    i = pl.program_id(0)

    @pl.when(i == 0)
    def _():
        sums[...] = jnp.zeros_like(sums)
        cnts[...] = jnp.zeros_like(cnts)

    inv0 = inv0_ref[...][:, 0:1]
    inv1 = inv1_ref[...][:, 0:1]
    z0 = (u2_ref[0] + y0_ref[...]) * inv0
    z1 = (u2_ref[1] + y1_ref[...]) * inv1
    a0 = jnp.dot(z0, w0_ref[...], preferred_element_type=_f32)
    a1 = jnp.dot(z1, w1_ref[...], preferred_element_type=_f32)
    xf = (a0 + a1) * 0.5
    o = jnp.dot(xf, wout_ref[...], preferred_element_type=_f32)
    b2d = batch_ref[0]
    iota_g = lax.broadcasted_iota(_i32, (_NGRAPH, _RB), 0)
    onehot_t = (b2d == iota_g).astype(_f32)
    sums[...] += jnp.dot(onehot_t, o, preferred_element_type=_f32)
    cnts[...] += jnp.sum(onehot_t, axis=1, keepdims=True)

    @pl.when(i == _GRID - 1)
    def _():
        out_ref[...] = sums[...] / jnp.maximum(cnts[...], 1.0) + bout_ref[...]


def _tc2(u2, y0, y1, inv0, inv1, W01, W11, W_out_pad, b_out_pad, batch_r):
    return pl.pallas_call(
        _tc2_body,
        grid=(_GRID,),
        in_specs=[
            pl.BlockSpec((_NC, _RB, _D), lambda i: (0, i, 0)),
            pl.BlockSpec((_RB, _D), lambda i: (i, 0)),
            pl.BlockSpec((_RB, _D), lambda i: (i, 0)),
            pl.BlockSpec((_RB, 16), lambda i: (i, 0)),
            pl.BlockSpec((_RB, 16), lambda i: (i, 0)),
            pl.BlockSpec((_D, _D), lambda i: (0, 0)),
            pl.BlockSpec((_D, _D), lambda i: (0, 0)),
            pl.BlockSpec((_D, _D), lambda i: (0, 0)),
            pl.BlockSpec((1, _D), lambda i: (0, 0)),
            pl.BlockSpec((1, 1, _RB), lambda i: (i, 0, 0)),
        ],
        out_specs=pl.BlockSpec((_NGRAPH, _D), lambda i: (0, 0)),
        out_shape=jax.ShapeDtypeStruct((_NGRAPH, _D), _f32),
        scratch_shapes=[
            pltpu.VMEM((_NGRAPH, _D), _f32),
            pltpu.VMEM((_NGRAPH, 1), _f32),
        ],
    )(u2, y0, y1, inv0, inv1, W01, W11, W_out_pad, b_out_pad, batch_r)


def kernel(x, edge_index_sc, edge_index_fc, batch,
           W0_0, W0_1, W1_0, W1_1, W_out, b_out):
    x_pad = jnp.zeros((_N_PAD, _D), _f32).at[:_N].set(x)

    def prep(ei):
        pad = jnp.full((_E_PAD - _E,), _N, _i32)
        src = jnp.concatenate([ei[0], pad]).reshape(
            _NS, _NGROUPS, _SGROUP, _SCHUNK)
        dst = jnp.concatenate([ei[1], pad]).reshape(
            _NS, _NGROUPS, _SGROUP, _SCHUNK)
        return src, dst

    src0, dst0 = prep(edge_index_sc)
    src1, dst1 = prep(edge_index_fc)
    src_all = jnp.stack([src0, src1])
    dst_all = jnp.stack([dst0, dst1])

    ones_rows = jnp.ones((_SCHUNK, _D), _f32)
    zrows = jnp.zeros((_RPT, _D), _f32)

    xt0, xt1, inv0, inv1, u = _sc_a(x_pad, src_all, dst_all, ones_rows, zrows)
    y0, y1 = _tc1(u, xt0, xt1, inv0, inv1, W0_0, W1_0)
    u2 = _sc_b(y0, y1, src_all, dst_all, zrows)

    W_out_pad = jnp.zeros((_D, _D), _f32).at[:, :_NCLASS].set(W_out)
    b_out_pad = jnp.zeros((1, _D), _f32).at[0, :_NCLASS].set(b_out)
    batch_r = jnp.concatenate(
        [batch, jnp.full((_N_PAD - _N,), _NGRAPH, _i32)]).reshape(_GRID, 1, _RB)

    outg = _tc2(u2, y0, y1, inv0, inv1, W0_1, W1_1,
                W_out_pad, b_out_pad, batch_r)
    return outg[:, :_NCLASS]


# speedup vs baseline: 1.0159x; 1.0159x over previous
import jax
import jax.numpy as jnp
from jax import lax
from jax.experimental import pallas as pl
from jax.experimental.pallas import tpu as pltpu
from jax.experimental.pallas import tpu_sc as plsc

_N = 10000
_E = 320000
_D = 128
_NCLASS = 10
_NGRAPH = 64

_NC = 2
_NS = 16
_N_PAD = 10240
_RPT = _N_PAD // _NS
_SCHUNK = 64
_SGROUP = 32
_NGROUPS = 10
_E_PAD = _NS * _NGROUPS * _SGROUP * _SCHUNK
_NBUF = 4
_SLAB = 64
_NSLAB = _RPT // _SLAB
_RB = 256
_GRID = _N_PAD // _RB

_f32 = jnp.float32
_i32 = jnp.int32


def _sc_mesh():
    return plsc.VectorSubcoreMesh(
        core_axis_name="c", subcore_axis_name="s",
        num_cores=_NC, num_subcores=_NS)


def _quake_rsqrt(d):
    i = lax.bitcast_convert_type(d, _i32)
    i = jnp.full((16,), 0x5F3759DF, _i32) - lax.shift_right_logical(
        i, jnp.full((16,), 1, _i32))
    y = lax.bitcast_convert_type(i, _f32)
    half = d * 0.5
    y = y * (1.5 - half * y * y)
    y = y * (1.5 - half * y * y)
    y = y * (1.5 - half * y * y)
    return y


def _edge_pipeline(table, h_hbm, src_v, dst_v, rows_v, gsems, ssems):
    gd = [None] * _SGROUP
    sd = [None] * _SGROUP

    def issue_gather(k):
        b = k % _NBUF
        gd[k] = pltpu.async_copy(
            h_hbm.at[src_v.at[k]], rows_v.at[b], gsems[b])

    issue_gather(0)
    for k in range(_SGROUP):
        b = k % _NBUF
        if k + 1 < _SGROUP:
            if k + 1 >= _NBUF:
                sd[k + 1 - _NBUF].wait()
            issue_gather(k + 1)
        gd[k].wait()
        sd[k] = pltpu.async_copy(
            rows_v.at[b], table.at[dst_v.at[k]], ssems[b], add=True)
    for k in range(_SGROUP - _NBUF, _SGROUP):
        sd[k].wait()


def _sc_a_body(x_hbm, src_hbm, dst_hbm, ones_hbm, zeros_hbm,
               xt0_hbm, xt1_hbm, inv0_hbm, inv1_hbm, u_hbm,
               table, src_v, dst_v, rows_v, invbuf,
               *sems):
    c = lax.axis_index("c")
    s = lax.axis_index("s")
    row0 = s * _RPT
    gsems = sems[:_NBUF]
    ssems = sems[_NBUF:]
    ones_v = rows_v.at[2]
    xbuf = rows_v.at[0]
    degbuf = rows_v.at[1]

    pltpu.sync_copy(zeros_hbm, table.at[pl.ds(row0, _RPT)])
    pltpu.sync_copy(ones_hbm, ones_v)
    plsc.subcore_barrier()

    def deg_group(g, carry):
        pltpu.sync_copy(dst_hbm.at[c, s, g], dst_v)
        sdg = [None] * _SGROUP
        for k in range(_SGROUP):
            if k >= _NBUF:
                sdg[k - _NBUF].wait()
            sdg[k] = pltpu.async_copy(
                ones_v, table.at[dst_v.at[k]], ssems[k % _NBUF], add=True)
        for k in range(_SGROUP - _NBUF, _SGROUP):
            sdg[k].wait()
        return carry

    lax.fori_loop(0, _NGROUPS, deg_group, 0)
    plsc.subcore_barrier()

    def run_scale(xt_hbm, inv_hbm):
        def slab(t, carry):
            r0 = row0 + t * _SLAB
            pltpu.sync_copy(x_hbm.at[pl.ds(r0, _SLAB)], xbuf)
            pltpu.sync_copy(table.at[pl.ds(r0, _SLAB)], degbuf)

            def row(r, carry2):
                d = degbuf[r, pl.ds(0, 16)] + 1.0
                inv = _quake_rsqrt(d)
                invbuf[r, pl.ds(0, 16)] = inv
                for q in range(_D // 16):
                    xbuf[r, pl.ds(q * 16, 16)] = (
                        xbuf[r, pl.ds(q * 16, 16)] * inv)
                return carry2

            lax.fori_loop(0, _SLAB, row, 0)
            pltpu.sync_copy(xbuf, xt_hbm.at[pl.ds(r0, _SLAB)])
            pltpu.sync_copy(invbuf, inv_hbm.at[pl.ds(r0, _SLAB)])
            return carry

        lax.fori_loop(0, _NSLAB, slab, 0)

    @pl.when(c == 0)
    def _():
        run_scale(xt0_hbm, inv0_hbm)

    @pl.when(c == 1)
    def _():
        run_scale(xt1_hbm, inv1_hbm)

    plsc.subcore_barrier()

    pltpu.sync_copy(zeros_hbm, table.at[pl.ds(row0, _RPT)])
    plsc.subcore_barrier()

    def run_edges(xt_hbm):
        def group(g, carry):
            pltpu.sync_copy(src_hbm.at[c, s, g], src_v)
            pltpu.sync_copy(dst_hbm.at[c, s, g], dst_v)
            _edge_pipeline(table, xt_hbm, src_v, dst_v, rows_v, gsems, ssems)
            return carry

        lax.fori_loop(0, _NGROUPS, group, 0)

    @pl.when(c == 0)
    def _():
        run_edges(xt0_hbm)

    @pl.when(c == 1)
    def _():
        run_edges(xt1_hbm)

    plsc.subcore_barrier()
    pltpu.sync_copy(table.at[pl.ds(row0, _RPT)],
                    u_hbm.at[c, pl.ds(row0, _RPT)])


def _sc_a(x_pad, src_all, dst_all, ones_rows, zrows):
    k = pl.kernel(
        _sc_a_body,
        out_type=[
            jax.ShapeDtypeStruct((_N_PAD, _D), _f32),
            jax.ShapeDtypeStruct((_N_PAD, _D), _f32),
            jax.ShapeDtypeStruct((_N_PAD, 16), _f32),
            jax.ShapeDtypeStruct((_N_PAD, 16), _f32),
            jax.ShapeDtypeStruct((_NC, _N_PAD, _D), _f32),
        ],
        mesh=_sc_mesh(),
        scratch_types=[
            pltpu.VMEM_SHARED((_N_PAD, _D), _f32),
            pltpu.VMEM((_SGROUP, _SCHUNK), _i32),
            pltpu.VMEM((_SGROUP, _SCHUNK), _i32),
            pltpu.VMEM((_NBUF, _SCHUNK, _D), _f32),
            pltpu.VMEM((_SLAB, 16), _f32),
        ] + [pltpu.SemaphoreType.DMA] * (2 * _NBUF),
    )
    return k(x_pad, src_all, dst_all, ones_rows, zrows)


def _sc_b_body(h0_hbm, h1_hbm, src_hbm, dst_hbm, zeros_hbm, out_hbm,
               table, src_v, dst_v, rows_v, *sems):
    c = lax.axis_index("c")
    s = lax.axis_index("s")
    row0 = s * _RPT
    gsems = sems[:_NBUF]
    ssems = sems[_NBUF:]
    pltpu.sync_copy(zeros_hbm, table.at[pl.ds(row0, _RPT)])
    plsc.subcore_barrier()

    def run_edges(h_hbm):
        def group(g, carry):
            pltpu.sync_copy(src_hbm.at[c, s, g], src_v)
            pltpu.sync_copy(dst_hbm.at[c, s, g], dst_v)
            _edge_pipeline(table, h_hbm, src_v, dst_v, rows_v, gsems, ssems)
            return carry

        lax.fori_loop(0, _NGROUPS, group, 0)

    @pl.when(c == 0)
    def _():
        run_edges(h0_hbm)

    @pl.when(c == 1)
    def _():
        run_edges(h1_hbm)

    plsc.subcore_barrier()
    pltpu.sync_copy(table.at[pl.ds(row0, _RPT)],
                    out_hbm.at[c, pl.ds(row0, _RPT)])


def _sc_b(h0, h1, src_all, dst_all, zrows):
    k = pl.kernel(
        _sc_b_body,
        out_type=jax.ShapeDtypeStruct((_NC, _N_PAD, _D), _f32),
        mesh=_sc_mesh(),
        scratch_types=[
            pltpu.VMEM_SHARED((_N_PAD, _D), _f32),
            pltpu.VMEM((_SGROUP, _SCHUNK), _i32),
            pltpu.VMEM((_SGROUP, _SCHUNK), _i32),
            pltpu.VMEM((_NBUF, _SCHUNK, _D), _f32),
        ] + [pltpu.SemaphoreType.DMA] * (2 * _NBUF),
    )
    return k(h0, h1, src_all, dst_all, zrows)


def _tc1_body(u_ref, xt0_ref, xt1_ref, inv0_ref, inv1_ref, w0_ref, w1_ref,
              y0_ref, y1_ref):
    inv0 = inv0_ref[...][:, 0:1]
    inv1 = inv1_ref[...][:, 0:1]
    v0 = (u_ref[0] + xt0_ref[...]) * inv0
    v1 = (u_ref[1] + xt1_ref[...]) * inv1
    a0 = jnp.maximum(jnp.dot(v0, w0_ref[...], preferred_element_type=_f32), 0.0)
    a1 = jnp.maximum(jnp.dot(v1, w1_ref[...], preferred_element_type=_f32), 0.0)
    y0_ref[...] = a0 * inv0
    y1_ref[...] = a1 * inv1


def _tc1(u, xt0, xt1, inv0, inv1, W00, W10):
    return pl.pallas_call(
        _tc1_body,
        grid=(_GRID,),
        in_specs=[
            pl.BlockSpec((_NC, _RB, _D), lambda i: (0, i, 0)),
            pl.BlockSpec((_RB, _D), lambda i: (i, 0)),
            pl.BlockSpec((_RB, _D), lambda i: (i, 0)),
            pl.BlockSpec((_RB, 16), lambda i: (i, 0)),
            pl.BlockSpec((_RB, 16), lambda i: (i, 0)),
            pl.BlockSpec((_D, _D), lambda i: (0, 0)),
            pl.BlockSpec((_D, _D), lambda i: (0, 0)),
        ],
        out_specs=[
            pl.BlockSpec((_RB, _D), lambda i: (i, 0)),
            pl.BlockSpec((_RB, _D), lambda i: (i, 0)),
        ],
        out_shape=[
            jax.ShapeDtypeStruct((_N_PAD, _D), _f32),
            jax.ShapeDtypeStruct((_N_PAD, _D), _f32),
        ],
    )(u, xt0, xt1, inv0, inv1, W00, W10)


def _tc2_body(u2_ref, y0_ref, y1_ref, inv0_ref, inv1_ref, w0_ref, w1_ref,
              wout_ref, bout_ref, batch_ref, out_ref, sums, cnts):
    i = pl.program_id(0)

    @pl.when(i == 0)
    def _():
        sums[...] = jnp.zeros_like(sums)
        cnts[...] = jnp.zeros_like(cnts)

    inv0 = inv0_ref[...][:, 0:1]
    inv1 = inv1_ref[...][:, 0:1]
    z0 = (u2_ref[0] + y0_ref[...]) * inv0
    z1 = (u2_ref[1] + y1_ref[...]) * inv1
    a0 = jnp.dot(z0, w0_ref[...], preferred_element_type=_f32)
    a1 = jnp.dot(z1, w1_ref[...], preferred_element_type=_f32)
    xf = (a0 + a1) * 0.5
    o = jnp.dot(xf, wout_ref[...], preferred_element_type=_f32)
    b2d = batch_ref[0]
    iota_g = lax.broadcasted_iota(_i32, (_NGRAPH, _RB), 0)
    onehot_t = (b2d == iota_g).astype(_f32)
    sums[...] += jnp.dot(onehot_t, o, preferred_element_type=_f32)
    cnts[...] += jnp.sum(onehot_t, axis=1, keepdims=True)

    @pl.when(i == _GRID - 1)
    def _():
        out_ref[...] = sums[...] / jnp.maximum(cnts[...], 1.0) + bout_ref[...]


def _tc2(u2, y0, y1, inv0, inv1, W01, W11, W_out_pad, b_out_pad, batch_r):
    return pl.pallas_call(
        _tc2_body,
        grid=(_GRID,),
        in_specs=[
            pl.BlockSpec((_NC, _RB, _D), lambda i: (0, i, 0)),
            pl.BlockSpec((_RB, _D), lambda i: (i, 0)),
            pl.BlockSpec((_RB, _D), lambda i: (i, 0)),
            pl.BlockSpec((_RB, 16), lambda i: (i, 0)),
            pl.BlockSpec((_RB, 16), lambda i: (i, 0)),
            pl.BlockSpec((_D, _D), lambda i: (0, 0)),
            pl.BlockSpec((_D, _D), lambda i: (0, 0)),
            pl.BlockSpec((_D, _D), lambda i: (0, 0)),
            pl.BlockSpec((1, _D), lambda i: (0, 0)),
            pl.BlockSpec((1, 1, _RB), lambda i: (i, 0, 0)),
        ],
        out_specs=pl.BlockSpec((_NGRAPH, _D), lambda i: (0, 0)),
        out_shape=jax.ShapeDtypeStruct((_NGRAPH, _D), _f32),
        scratch_shapes=[
            pltpu.VMEM((_NGRAPH, _D), _f32),
            pltpu.VMEM((_NGRAPH, 1), _f32),
        ],
    )(u2, y0, y1, inv0, inv1, W01, W11, W_out_pad, b_out_pad, batch_r)


def kernel(x, edge_index_sc, edge_index_fc, batch,
           W0_0, W0_1, W1_0, W1_1, W_out, b_out):
    x_pad = jnp.zeros((_N_PAD, _D), _f32).at[:_N].set(x)

    def prep(ei):
        pad = jnp.full((_E_PAD - _E,), _N, _i32)
        src = jnp.concatenate([ei[0], pad]).reshape(
            _NS, _NGROUPS, _SGROUP, _SCHUNK)
        dst = jnp.concatenate([ei[1], pad]).reshape(
            _NS, _NGROUPS, _SGROUP, _SCHUNK)
        return src, dst

    src0, dst0 = prep(edge_index_sc)
    src1, dst1 = prep(edge_index_fc)
    src_all = jnp.stack([src0, src1])
    dst_all = jnp.stack([dst0, dst1])

    ones_rows = jnp.ones((_SCHUNK, _D), _f32)
    zrows = jnp.zeros((_RPT, _D), _f32)

    xt0, xt1, inv0, inv1, u = _sc_a(x_pad, src_all, dst_all, ones_rows, zrows)
    y0, y1 = _tc1(u, xt0, xt1, inv0, inv1, W0_0, W1_0)
    u2 = _sc_b(y0, y1, src_all, dst_all, zrows)

    W_out_pad = jnp.zeros((_D, _D), _f32).at[:, :_NCLASS].set(W_out)
    b_out_pad = jnp.zeros((1, _D), _f32).at[0, :_NCLASS].set(b_out)
    batch_r = jnp.concatenate(
        [batch, jnp.full((_N_PAD - _N,), _NGRAPH, _i32)]).reshape(_GRID, 1, _RB)

    outg = _tc2(u2, y0, y1, inv0, inv1, W0_1, W1_1,
                W_out_pad, b_out_pad, batch_r)
    return outg[:, :_NCLASS]


# speedup vs baseline: 1.0995x; 1.0823x over previous
import jax
import jax.numpy as jnp
from jax import lax
from jax.experimental import pallas as pl
from jax.experimental.pallas import tpu as pltpu
from jax.experimental.pallas import tpu_sc as plsc

_N = 10000
_E = 320000
_D = 128
_NCLASS = 10
_NGRAPH = 64

_NC = 2
_NS = 16
_N_PAD = 10240
_RPT = _N_PAD // _NS
_CHUNK = 128
_GROUP = 16
_NGROUPS = 10
_NCHUNK = _GROUP * _NGROUPS
_E_PAD = _NS * _NCHUNK * _CHUNK
_SCHUNK = 64
_SGROUP = 32
_NBUF = 4
_DEGW = 128
_RB = 256
_GRID = _N_PAD // _RB

_f32 = jnp.float32
_i32 = jnp.int32


def _sc_mesh():
    return plsc.VectorSubcoreMesh(
        core_axis_name="c", subcore_axis_name="s",
        num_cores=_NC, num_subcores=_NS)


def _sc_deg_body(dst_hbm, ones_hbm, zeros_hbm, out_hbm, deg_acc, dst_v, ones_v,
                 sem0, sem1, sem2, sem3):
    c = lax.axis_index("c")
    s = lax.axis_index("s")
    row0 = s * _RPT
    sems = (sem0, sem1, sem2, sem3)
    pltpu.sync_copy(zeros_hbm, deg_acc.at[pl.ds(row0, _RPT)])
    pltpu.sync_copy(ones_hbm, ones_v)
    plsc.subcore_barrier()

    def group(g, carry):
        pltpu.sync_copy(dst_hbm.at[c, s, g], dst_v)
        sd = [None] * _GROUP
        for k in range(_GROUP):
            if k >= 4:
                sd[k - 4].wait()
            sd[k] = pltpu.async_copy(
                ones_v, deg_acc.at[dst_v.at[k]], sems[k % 4], add=True)
        for k in range(_GROUP - 4, _GROUP):
            sd[k].wait()
        return carry

    lax.fori_loop(0, _NGROUPS, group, 0)
    plsc.subcore_barrier()
    pltpu.sync_copy(deg_acc.at[pl.ds(row0, _RPT)],
                    out_hbm.at[c, pl.ds(row0, _RPT)])


def _sc_deg(dst_all, ones8, zeros8):
    k = pl.kernel(
        _sc_deg_body,
        out_type=jax.ShapeDtypeStruct((_NC, _N_PAD, _DEGW), _f32),
        mesh=_sc_mesh(),
        scratch_types=[
            pltpu.VMEM_SHARED((_N_PAD, _DEGW), _f32),
            pltpu.VMEM((_GROUP, _CHUNK), _i32),
            pltpu.VMEM((_CHUNK, _DEGW), _f32),
            pltpu.SemaphoreType.DMA,
            pltpu.SemaphoreType.DMA,
            pltpu.SemaphoreType.DMA,
            pltpu.SemaphoreType.DMA,
        ],
    )
    return k(dst_all, ones8, zeros8)


def _sc_scatter_body(h0_hbm, h1_hbm, src_hbm, dst_hbm, zeros_hbm, out_hbm,
                     acc, src_v, dst_v, rows_v, *sems):
    c = lax.axis_index("c")
    s = lax.axis_index("s")
    row0 = s * _RPT
    gsems = sems[:_NBUF]
    ssems = sems[_NBUF:]
    pltpu.sync_copy(zeros_hbm, acc.at[pl.ds(row0, _RPT)])
    plsc.subcore_barrier()

    def run(h_hbm):
        def group(g, carry):
            pltpu.sync_copy(src_hbm.at[c, s, g], src_v)
            pltpu.sync_copy(dst_hbm.at[c, s, g], dst_v)
            gd = [None] * _SGROUP
            sd = [None] * _SGROUP

            def issue_gather(k):
                b = k % _NBUF
                gd[k] = pltpu.async_copy(
                    h_hbm.at[src_v.at[k]], rows_v.at[b], gsems[b])

            issue_gather(0)
            for k in range(_SGROUP):
                b = k % _NBUF
                if k + 1 < _SGROUP:
                    if k + 1 >= _NBUF:
                        sd[k + 1 - _NBUF].wait()
                    issue_gather(k + 1)
                gd[k].wait()
                sd[k] = pltpu.async_copy(
                    rows_v.at[b], acc.at[dst_v.at[k]], ssems[b], add=True)
            for k in range(_SGROUP - _NBUF, _SGROUP):
                sd[k].wait()
            return carry

        lax.fori_loop(0, _NGROUPS, group, 0)

    @pl.when(c == 0)
    def _():
        run(h0_hbm)

    @pl.when(c == 1)
    def _():
        run(h1_hbm)

    plsc.subcore_barrier()
    pltpu.sync_copy(acc.at[pl.ds(row0, _RPT)],
                    out_hbm.at[c, pl.ds(row0, _RPT)])


def _sc_scatter(h0, h1, src_all, dst_all, zrows):
    k = pl.kernel(
        _sc_scatter_body,
        out_type=jax.ShapeDtypeStruct((_NC, _N_PAD, _D), _f32),
        mesh=_sc_mesh(),
        scratch_types=[
            pltpu.VMEM_SHARED((_N_PAD, _D), _f32),
            pltpu.VMEM((_SGROUP, _SCHUNK), _i32),
            pltpu.VMEM((_SGROUP, _SCHUNK), _i32),
            pltpu.VMEM((_NBUF, _SCHUNK, _D), _f32),
        ] + [pltpu.SemaphoreType.DMA] * (2 * _NBUF),
    )
    return k(h0, h1, src_all, dst_all, zrows)


def _tc1_body(x_ref, w0_ref, w1_ref, deg_ref, h0_ref, h1_ref):
    xb = x_ref[...]
    inv0 = lax.rsqrt(deg_ref[0] + 1.0)
    inv1 = lax.rsqrt(deg_ref[1] + 1.0)
    h0_ref[...] = jnp.dot(xb, w0_ref[...], preferred_element_type=_f32) * inv0
    h1_ref[...] = jnp.dot(xb, w1_ref[...], preferred_element_type=_f32) * inv1


def _tc1(x_pad, W00, W10, deg_col):
    return pl.pallas_call(
        _tc1_body,
        grid=(_GRID,),
        in_specs=[
            pl.BlockSpec((_RB, _D), lambda i: (i, 0)),
            pl.BlockSpec((_D, _D), lambda i: (0, 0)),
            pl.BlockSpec((_D, _D), lambda i: (0, 0)),
            pl.BlockSpec((_NC, _RB, 1), lambda i: (0, i, 0)),
        ],
        out_specs=[
            pl.BlockSpec((_RB, _D), lambda i: (i, 0)),
            pl.BlockSpec((_RB, _D), lambda i: (i, 0)),
        ],
        out_shape=[
            jax.ShapeDtypeStruct((_N_PAD, _D), _f32),
            jax.ShapeDtypeStruct((_N_PAD, _D), _f32),
        ],
    )(x_pad, W00, W10, deg_col)


def _tc2_body(s_ref, h0_ref, h1_ref, deg_ref, w0_ref, w1_ref, o0_ref, o1_ref):
    inv0 = lax.rsqrt(deg_ref[0] + 1.0)
    inv1 = lax.rsqrt(deg_ref[1] + 1.0)
    a0 = jnp.maximum((s_ref[0] + h0_ref[...]) * inv0, 0.0)
    a1 = jnp.maximum((s_ref[1] + h1_ref[...]) * inv1, 0.0)
    o0_ref[...] = jnp.dot(a0, w0_ref[...], preferred_element_type=_f32) * inv0
    o1_ref[...] = jnp.dot(a1, w1_ref[...], preferred_element_type=_f32) * inv1


def _tc2(s1, h0, h1, deg_col, W01, W11):
    return pl.pallas_call(
        _tc2_body,
        grid=(_GRID,),
        in_specs=[
            pl.BlockSpec((_NC, _RB, _D), lambda i: (0, i, 0)),
            pl.BlockSpec((_RB, _D), lambda i: (i, 0)),
            pl.BlockSpec((_RB, _D), lambda i: (i, 0)),
            pl.BlockSpec((_NC, _RB, 1), lambda i: (0, i, 0)),
            pl.BlockSpec((_D, _D), lambda i: (0, 0)),
            pl.BlockSpec((_D, _D), lambda i: (0, 0)),
        ],
        out_specs=[
            pl.BlockSpec((_RB, _D), lambda i: (i, 0)),
            pl.BlockSpec((_RB, _D), lambda i: (i, 0)),
        ],
        out_shape=[
            jax.ShapeDtypeStruct((_N_PAD, _D), _f32),
            jax.ShapeDtypeStruct((_N_PAD, _D), _f32),
        ],
    )(s1, h0, h1, deg_col, W01, W11)


def _tc3_body(s_ref, h0_ref, h1_ref, deg_ref, wout_ref, bout_ref, batch_ref,
              out_ref, sums, cnts):
    i = pl.program_id(0)

    @pl.when(i == 0)
    def _():
        sums[...] = jnp.zeros_like(sums)
        cnts[...] = jnp.zeros_like(cnts)

    inv0 = lax.rsqrt(deg_ref[0] + 1.0)
    inv1 = lax.rsqrt(deg_ref[1] + 1.0)
    a0 = (s_ref[0] + h0_ref[...]) * inv0
    a1 = (s_ref[1] + h1_ref[...]) * inv1
    xf = (a0 + a1) * 0.5
    o = jnp.dot(xf, wout_ref[...], preferred_element_type=_f32)
    b2d = batch_ref[0]
    iota_g = lax.broadcasted_iota(_i32, (_NGRAPH, _RB), 0)
    onehot_t = (b2d == iota_g).astype(_f32)
    sums[...] += jnp.dot(onehot_t, o, preferred_element_type=_f32)
    cnts[...] += jnp.sum(onehot_t, axis=1, keepdims=True)

    @pl.when(i == _GRID - 1)
    def _():
        out_ref[...] = sums[...] / jnp.maximum(cnts[...], 1.0) + bout_ref[...]


def _tc3(s2, h0b, h1b, deg_col, W_out_pad, b_out_pad, batch_r):
    return pl.pallas_call(
        _tc3_body,
        grid=(_GRID,),
        in_specs=[
            pl.BlockSpec((_NC, _RB, _D), lambda i: (0, i, 0)),
            pl.BlockSpec((_RB, _D), lambda i: (i, 0)),
            pl.BlockSpec((_RB, _D), lambda i: (i, 0)),
            pl.BlockSpec((_NC, _RB, 1), lambda i: (0, i, 0)),
            pl.BlockSpec((_D, _D), lambda i: (0, 0)),
            pl.BlockSpec((1, _D), lambda i: (0, 0)),
            pl.BlockSpec((1, 1, _RB), lambda i: (i, 0, 0)),
        ],
        out_specs=pl.BlockSpec((_NGRAPH, _D), lambda i: (0, 0)),
        out_shape=jax.ShapeDtypeStruct((_NGRAPH, _D), _f32),
        scratch_shapes=[
            pltpu.VMEM((_NGRAPH, _D), _f32),
            pltpu.VMEM((_NGRAPH, 1), _f32),
        ],
    )(s2, h0b, h1b, deg_col, W_out_pad, b_out_pad, batch_r)


def kernel(x, edge_index_sc, edge_index_fc, batch,
           W0_0, W0_1, W1_0, W1_1, W_out, b_out):
    x_pad = jnp.zeros((_N_PAD, _D), _f32).at[:_N].set(x)

    def prep(ei):
        pad = jnp.full((_E_PAD - _E,), _N, _i32)
        src = jnp.concatenate([ei[0], pad]).reshape(_NS, _NGROUPS, _GROUP, _CHUNK)
        dst = jnp.concatenate([ei[1], pad]).reshape(_NS, _NGROUPS, _GROUP, _CHUNK)
        return src, dst

    src0, dst0 = prep(edge_index_sc)
    src1, dst1 = prep(edge_index_fc)
    src_all = jnp.stack([src0, src1])
    dst_all = jnp.stack([dst0, dst1])

    ones128 = jnp.ones((_CHUNK, _DEGW), _f32)
    zrows = jnp.zeros((_RPT, _D), _f32)

    deg_raw = _sc_deg(dst_all, ones128, zrows)
    deg_col = deg_raw[:, :, 0:1]

    src_all_s = src_all.reshape(_NC, _NS, _NGROUPS, _SGROUP, _SCHUNK)
    dst_all_s = dst_all.reshape(_NC, _NS, _NGROUPS, _SGROUP, _SCHUNK)

    h0, h1 = _tc1(x_pad, W0_0, W1_0, deg_col)
    s1 = _sc_scatter(h0, h1, src_all_s, dst_all_s, zrows)
    h0b, h1b = _tc2(s1, h0, h1, deg_col, W0_1, W1_1)
    s2 = _sc_scatter(h0b, h1b, src_all_s, dst_all_s, zrows)

    W_out_pad = jnp.zeros((_D, _D), _f32).at[:, :_NCLASS].set(W_out)
    b_out_pad = jnp.zeros((1, _D), _f32).at[0, :_NCLASS].set(b_out)
    batch_r = jnp.concatenate(
        [batch, jnp.full((_N_PAD - _N,), _NGRAPH, _i32)]).reshape(_GRID, 1, _RB)

    outg = _tc3(s2, h0b, h1b, deg_col, W_out_pad, b_out_pad, batch_r)
    return outg[:, :_NCLASS]


# speedup vs baseline: 1.1134x; 1.0126x over previous
import jax
import jax.numpy as jnp
from jax import lax
from jax.experimental import pallas as pl
from jax.experimental.pallas import tpu as pltpu
from jax.experimental.pallas import tpu_sc as plsc

_N = 10000
_E = 320000
_D = 128
_NCLASS = 10
_NGRAPH = 64

_NC = 2
_NS = 16
_N_PAD = 10240
_RPT = _N_PAD // _NS
_CHUNK = 128
_GROUP = 16
_NGROUPS = 10
_NCHUNK = _GROUP * _NGROUPS
_E_PAD = _NS * _NCHUNK * _CHUNK
_SCHUNK = 64
_SGROUP = 64
_SNGROUPS = 5
_NBUF = 4
_DEGW = 128
_RB = 256
_GRID = _N_PAD // _RB

_f32 = jnp.float32
_i32 = jnp.int32


def _sc_mesh():
    return plsc.VectorSubcoreMesh(
        core_axis_name="c", subcore_axis_name="s",
        num_cores=_NC, num_subcores=_NS)


def _sc_deg_body(dst_hbm, ones_hbm, zeros_hbm, out_hbm, deg_acc, dst_v, ones_v,
                 sem0, sem1, sem2, sem3):
    c = lax.axis_index("c")
    s = lax.axis_index("s")
    row0 = s * _RPT
    sems = (sem0, sem1, sem2, sem3)
    pltpu.sync_copy(zeros_hbm, deg_acc.at[pl.ds(row0, _RPT)])
    pltpu.sync_copy(ones_hbm, ones_v)
    plsc.subcore_barrier()

    def group(g, carry):
        pltpu.sync_copy(dst_hbm.at[c, s, g], dst_v)
        sd = [None] * _GROUP
        for k in range(_GROUP):
            if k >= 4:
                sd[k - 4].wait()
            sd[k] = pltpu.async_copy(
                ones_v, deg_acc.at[dst_v.at[k]], sems[k % 4], add=True)
        for k in range(_GROUP - 4, _GROUP):
            sd[k].wait()
        return carry

    lax.fori_loop(0, _NGROUPS, group, 0)
    plsc.subcore_barrier()
    pltpu.sync_copy(deg_acc.at[pl.ds(row0, _RPT)],
                    out_hbm.at[c, pl.ds(row0, _RPT)])


def _sc_deg(dst_all, ones8, zeros8):
    k = pl.kernel(
        _sc_deg_body,
        out_type=jax.ShapeDtypeStruct((_NC, _N_PAD, _DEGW), _f32),
        mesh=_sc_mesh(),
        scratch_types=[
            pltpu.VMEM_SHARED((_N_PAD, _DEGW), _f32),
            pltpu.VMEM((_GROUP, _CHUNK), _i32),
            pltpu.VMEM((_CHUNK, _DEGW), _f32),
            pltpu.SemaphoreType.DMA,
            pltpu.SemaphoreType.DMA,
            pltpu.SemaphoreType.DMA,
            pltpu.SemaphoreType.DMA,
        ],
    )
    return k(dst_all, ones8, zeros8)


def _sc_scatter_body(h0_hbm, h1_hbm, src_hbm, dst_hbm, zeros_hbm, out_hbm,
                     acc, src_v, dst_v, rows_v, *sems):
    c = lax.axis_index("c")
    s = lax.axis_index("s")
    row0 = s * _RPT
    gsems = sems[:_NBUF]
    ssems = sems[_NBUF:]
    pltpu.sync_copy(zeros_hbm, acc.at[pl.ds(row0, _RPT)])
    plsc.subcore_barrier()

    def run(h_hbm):
        def group(g, carry):
            pltpu.sync_copy(src_hbm.at[c, s, g], src_v)
            pltpu.sync_copy(dst_hbm.at[c, s, g], dst_v)
            gd = [None] * _SGROUP
            sd = [None] * _SGROUP

            def issue_gather(k):
                b = k % _NBUF
                gd[k] = pltpu.async_copy(
                    h_hbm.at[src_v.at[k]], rows_v.at[b], gsems[b])

            issue_gather(0)
            for k in range(_SGROUP):
                b = k % _NBUF
                if k + 1 < _SGROUP:
                    if k + 1 >= _NBUF:
                        sd[k + 1 - _NBUF].wait()
                    issue_gather(k + 1)
                gd[k].wait()
                sd[k] = pltpu.async_copy(
                    rows_v.at[b], acc.at[dst_v.at[k]], ssems[b], add=True)
            for k in range(_SGROUP - _NBUF, _SGROUP):
                sd[k].wait()
            return carry

        lax.fori_loop(0, _SNGROUPS, group, 0)

    @pl.when(c == 0)
    def _():
        run(h0_hbm)

    @pl.when(c == 1)
    def _():
        run(h1_hbm)

    plsc.subcore_barrier()
    pltpu.sync_copy(acc.at[pl.ds(row0, _RPT)],
                    out_hbm.at[c, pl.ds(row0, _RPT)])


def _sc_scatter(h0, h1, src_all, dst_all, zrows):
    k = pl.kernel(
        _sc_scatter_body,
        out_type=jax.ShapeDtypeStruct((_NC, _N_PAD, _D), _f32),
        mesh=_sc_mesh(),
        scratch_types=[
            pltpu.VMEM_SHARED((_N_PAD, _D), _f32),
            pltpu.VMEM((_SGROUP, _SCHUNK), _i32),
            pltpu.VMEM((_SGROUP, _SCHUNK), _i32),
            pltpu.VMEM((_NBUF, _SCHUNK, _D), _f32),
        ] + [pltpu.SemaphoreType.DMA] * (2 * _NBUF),
    )
    return k(h0, h1, src_all, dst_all, zrows)


def _tc1_body(x_ref, w0_ref, w1_ref, deg_ref, h0_ref, h1_ref):
    xb = x_ref[...]
    inv0 = lax.rsqrt(deg_ref[0] + 1.0)
    inv1 = lax.rsqrt(deg_ref[1] + 1.0)
    h0_ref[...] = jnp.dot(xb, w0_ref[...], preferred_element_type=_f32) * inv0
    h1_ref[...] = jnp.dot(xb, w1_ref[...], preferred_element_type=_f32) * inv1


def _tc1(x_pad, W00, W10, deg_col):
    return pl.pallas_call(
        _tc1_body,
        grid=(_GRID,),
        in_specs=[
            pl.BlockSpec((_RB, _D), lambda i: (i, 0)),
            pl.BlockSpec((_D, _D), lambda i: (0, 0)),
            pl.BlockSpec((_D, _D), lambda i: (0, 0)),
            pl.BlockSpec((_NC, _RB, 1), lambda i: (0, i, 0)),
        ],
        out_specs=[
            pl.BlockSpec((_RB, _D), lambda i: (i, 0)),
            pl.BlockSpec((_RB, _D), lambda i: (i, 0)),
        ],
        out_shape=[
            jax.ShapeDtypeStruct((_N_PAD, _D), _f32),
            jax.ShapeDtypeStruct((_N_PAD, _D), _f32),
        ],
    )(x_pad, W00, W10, deg_col)


def _tc2_body(s_ref, h0_ref, h1_ref, deg_ref, w0_ref, w1_ref, o0_ref, o1_ref):
    inv0 = lax.rsqrt(deg_ref[0] + 1.0)
    inv1 = lax.rsqrt(deg_ref[1] + 1.0)
    a0 = jnp.maximum((s_ref[0] + h0_ref[...]) * inv0, 0.0)
    a1 = jnp.maximum((s_ref[1] + h1_ref[...]) * inv1, 0.0)
    o0_ref[...] = jnp.dot(a0, w0_ref[...], preferred_element_type=_f32) * inv0
    o1_ref[...] = jnp.dot(a1, w1_ref[...], preferred_element_type=_f32) * inv1


def _tc2(s1, h0, h1, deg_col, W01, W11):
    return pl.pallas_call(
        _tc2_body,
        grid=(_GRID,),
        in_specs=[
            pl.BlockSpec((_NC, _RB, _D), lambda i: (0, i, 0)),
            pl.BlockSpec((_RB, _D), lambda i: (i, 0)),
            pl.BlockSpec((_RB, _D), lambda i: (i, 0)),
            pl.BlockSpec((_NC, _RB, 1), lambda i: (0, i, 0)),
            pl.BlockSpec((_D, _D), lambda i: (0, 0)),
            pl.BlockSpec((_D, _D), lambda i: (0, 0)),
        ],
        out_specs=[
            pl.BlockSpec((_RB, _D), lambda i: (i, 0)),
            pl.BlockSpec((_RB, _D), lambda i: (i, 0)),
        ],
        out_shape=[
            jax.ShapeDtypeStruct((_N_PAD, _D), _f32),
            jax.ShapeDtypeStruct((_N_PAD, _D), _f32),
        ],
    )(s1, h0, h1, deg_col, W01, W11)


def _tc3_body(s_ref, h0_ref, h1_ref, deg_ref, wout_ref, bout_ref, batch_ref,
              out_ref, sums, cnts):
    i = pl.program_id(0)

    @pl.when(i == 0)
    def _():
        sums[...] = jnp.zeros_like(sums)
        cnts[...] = jnp.zeros_like(cnts)

    inv0 = lax.rsqrt(deg_ref[0] + 1.0)
    inv1 = lax.rsqrt(deg_ref[1] + 1.0)
    a0 = (s_ref[0] + h0_ref[...]) * inv0
    a1 = (s_ref[1] + h1_ref[...]) * inv1
    xf = (a0 + a1) * 0.5
    o = jnp.dot(xf, wout_ref[...], preferred_element_type=_f32)
    b2d = batch_ref[0]
    iota_g = lax.broadcasted_iota(_i32, (_NGRAPH, _RB), 0)
    onehot_t = (b2d == iota_g).astype(_f32)
    sums[...] += jnp.dot(onehot_t, o, preferred_element_type=_f32)
    cnts[...] += jnp.sum(onehot_t, axis=1, keepdims=True)

    @pl.when(i == _GRID - 1)
    def _():
        out_ref[...] = sums[...] / jnp.maximum(cnts[...], 1.0) + bout_ref[...]


def _tc3(s2, h0b, h1b, deg_col, W_out_pad, b_out_pad, batch_r):
    return pl.pallas_call(
        _tc3_body,
        grid=(_GRID,),
        in_specs=[
            pl.BlockSpec((_NC, _RB, _D), lambda i: (0, i, 0)),
            pl.BlockSpec((_RB, _D), lambda i: (i, 0)),
            pl.BlockSpec((_RB, _D), lambda i: (i, 0)),
            pl.BlockSpec((_NC, _RB, 1), lambda i: (0, i, 0)),
            pl.BlockSpec((_D, _D), lambda i: (0, 0)),
            pl.BlockSpec((1, _D), lambda i: (0, 0)),
            pl.BlockSpec((1, 1, _RB), lambda i: (i, 0, 0)),
        ],
        out_specs=pl.BlockSpec((_NGRAPH, _D), lambda i: (0, 0)),
        out_shape=jax.ShapeDtypeStruct((_NGRAPH, _D), _f32),
        scratch_shapes=[
            pltpu.VMEM((_NGRAPH, _D), _f32),
            pltpu.VMEM((_NGRAPH, 1), _f32),
        ],
    )(s2, h0b, h1b, deg_col, W_out_pad, b_out_pad, batch_r)


def kernel(x, edge_index_sc, edge_index_fc, batch,
           W0_0, W0_1, W1_0, W1_1, W_out, b_out):
    x_pad = jnp.zeros((_N_PAD, _D), _f32).at[:_N].set(x)

    def prep(ei):
        pad = jnp.full((_E_PAD - _E,), _N, _i32)
        src = jnp.concatenate([ei[0], pad]).reshape(_NS, _NGROUPS, _GROUP, _CHUNK)
        dst = jnp.concatenate([ei[1], pad]).reshape(_NS, _NGROUPS, _GROUP, _CHUNK)
        return src, dst

    src0, dst0 = prep(edge_index_sc)
    src1, dst1 = prep(edge_index_fc)
    src_all = jnp.stack([src0, src1])
    dst_all = jnp.stack([dst0, dst1])

    ones128 = jnp.ones((_CHUNK, _DEGW), _f32)
    zrows = jnp.zeros((_RPT, _D), _f32)

    deg_raw = _sc_deg(dst_all, ones128, zrows)
    deg_col = deg_raw[:, :, 0:1]

    src_all_s = src_all.reshape(_NC, _NS, _SNGROUPS, _SGROUP, _SCHUNK)
    dst_all_s = dst_all.reshape(_NC, _NS, _SNGROUPS, _SGROUP, _SCHUNK)

    h0, h1 = _tc1(x_pad, W0_0, W1_0, deg_col)
    s1 = _sc_scatter(h0, h1, src_all_s, dst_all_s, zrows)
    h0b, h1b = _tc2(s1, h0, h1, deg_col, W0_1, W1_1)
    s2 = _sc_scatter(h0b, h1b, src_all_s, dst_all_s, zrows)

    W_out_pad = jnp.zeros((_D, _D), _f32).at[:, :_NCLASS].set(W_out)
    b_out_pad = jnp.zeros((1, _D), _f32).at[0, :_NCLASS].set(b_out)
    batch_r = jnp.concatenate(
        [batch, jnp.full((_N_PAD - _N,), _NGRAPH, _i32)]).reshape(_GRID, 1, _RB)

    outg = _tc3(s2, h0b, h1b, deg_col, W_out_pad, b_out_pad, batch_r)
    return outg[:, :_NCLASS]


# speedup vs baseline: 1.1546x; 1.0370x over previous
import jax
import jax.numpy as jnp
from jax import lax
from jax.experimental import pallas as pl
from jax.experimental.pallas import tpu as pltpu
from jax.experimental.pallas import tpu_sc as plsc

_N = 10000
_E = 320000
_D = 128
_NCLASS = 10
_NGRAPH = 64

_NC = 2
_NS = 16
_N_PAD = 10240
_RPT = _N_PAD // _NS
_CHUNK = 128
_GROUP = 16
_NGROUPS = 10
_NCHUNK = _GROUP * _NGROUPS
_E_PAD = _NS * _NCHUNK * _CHUNK
_SCHUNK = 64
_SGROUP = 64
_SNGROUPS = 5
_NBUF = 4
_DEGW = 128
_RB = 256
_GRID = _N_PAD // _RB

_f32 = jnp.float32
_i32 = jnp.int32


def _sc_mesh():
    return plsc.VectorSubcoreMesh(
        core_axis_name="c", subcore_axis_name="s",
        num_cores=_NC, num_subcores=_NS)


def _sc_deg_body(dst_hbm, ones_hbm, zeros_hbm, out_hbm, deg_acc, dst_v, ones_v,
                 sem0, sem1, sem2, sem3):
    c = lax.axis_index("c")
    s = lax.axis_index("s")
    row0 = s * _RPT
    sems = (sem0, sem1, sem2, sem3)
    pltpu.sync_copy(zeros_hbm, deg_acc.at[pl.ds(row0, _RPT)])
    pltpu.sync_copy(ones_hbm, ones_v)
    plsc.subcore_barrier()

    def group(g, carry):
        pltpu.sync_copy(dst_hbm.at[c, s, g], dst_v)
        sd = [None] * _GROUP
        for k in range(_GROUP):
            if k >= 4:
                sd[k - 4].wait()
            sd[k] = pltpu.async_copy(
                ones_v, deg_acc.at[dst_v.at[k]], sems[k % 4], add=True)
        for k in range(_GROUP - 4, _GROUP):
            sd[k].wait()
        return carry

    lax.fori_loop(0, _NGROUPS, group, 0)
    plsc.subcore_barrier()
    pltpu.sync_copy(deg_acc.at[pl.ds(row0, _RPT)],
                    out_hbm.at[c, pl.ds(row0, _RPT)])


def _sc_deg(dst_all, ones8, zeros8):
    k = pl.kernel(
        _sc_deg_body,
        out_type=jax.ShapeDtypeStruct((_NC, _N_PAD, _DEGW), _f32),
        mesh=_sc_mesh(),
        scratch_types=[
            pltpu.VMEM_SHARED((_N_PAD, _DEGW), _f32),
            pltpu.VMEM((_GROUP, _CHUNK), _i32),
            pltpu.VMEM((_CHUNK, _DEGW), _f32),
            pltpu.SemaphoreType.DMA,
            pltpu.SemaphoreType.DMA,
            pltpu.SemaphoreType.DMA,
            pltpu.SemaphoreType.DMA,
        ],
    )
    return k(dst_all, ones8, zeros8)


def _sc_scatter_body(h0_hbm, h1_hbm, src_hbm, dst_hbm, zeros_hbm, out_hbm,
                     acc, src_v, dst_v, rows_v, *sems):
    c = lax.axis_index("c")
    s = lax.axis_index("s")
    row0 = s * _RPT
    gsems = sems[:_NBUF]
    ssems = sems[_NBUF:]
    pltpu.sync_copy(zeros_hbm, acc.at[pl.ds(row0, _RPT)])
    plsc.subcore_barrier()

    def run(h_hbm):
        def group(g, carry):
            pltpu.sync_copy(src_hbm.at[c, s, g], src_v)
            pltpu.sync_copy(dst_hbm.at[c, s, g], dst_v)
            gd = [None] * _SGROUP
            sd = [None] * _SGROUP

            def issue_gather(k):
                b = k % _NBUF
                gd[k] = pltpu.async_copy(
                    h_hbm.at[src_v.at[k]], rows_v.at[b], gsems[b])

            for k in range(_NBUF - 1):
                issue_gather(k)
            for k in range(_SGROUP):
                b = k % _NBUF
                j = k + _NBUF - 1
                if j < _SGROUP:
                    if j >= _NBUF:
                        sd[j - _NBUF].wait()
                    issue_gather(j)
                gd[k].wait()
                sd[k] = pltpu.async_copy(
                    rows_v.at[b], acc.at[dst_v.at[k]], ssems[b], add=True)
            for k in range(_SGROUP - _NBUF, _SGROUP):
                sd[k].wait()
            return carry

        lax.fori_loop(0, _SNGROUPS, group, 0)

    @pl.when(c == 0)
    def _():
        run(h0_hbm)

    @pl.when(c == 1)
    def _():
        run(h1_hbm)

    plsc.subcore_barrier()
    pltpu.sync_copy(acc.at[pl.ds(row0, _RPT)],
                    out_hbm.at[c, pl.ds(row0, _RPT)])


def _sc_scatter(h0, h1, src_all, dst_all, zrows):
    k = pl.kernel(
        _sc_scatter_body,
        out_type=jax.ShapeDtypeStruct((_NC, _N_PAD, _D), _f32),
        mesh=_sc_mesh(),
        scratch_types=[
            pltpu.VMEM_SHARED((_N_PAD, _D), _f32),
            pltpu.VMEM((_SGROUP, _SCHUNK), _i32),
            pltpu.VMEM((_SGROUP, _SCHUNK), _i32),
            pltpu.VMEM((_NBUF, _SCHUNK, _D), _f32),
        ] + [pltpu.SemaphoreType.DMA] * (2 * _NBUF),
    )
    return k(h0, h1, src_all, dst_all, zrows)


def _tc1_body(x_ref, w0_ref, w1_ref, deg_ref, h0_ref, h1_ref):
    xb = x_ref[...]
    inv0 = lax.rsqrt(deg_ref[0] + 1.0)
    inv1 = lax.rsqrt(deg_ref[1] + 1.0)
    h0_ref[...] = jnp.dot(xb, w0_ref[...], preferred_element_type=_f32) * inv0
    h1_ref[...] = jnp.dot(xb, w1_ref[...], preferred_element_type=_f32) * inv1


def _tc1(x_pad, W00, W10, deg_col):
    return pl.pallas_call(
        _tc1_body,
        grid=(_GRID,),
        in_specs=[
            pl.BlockSpec((_RB, _D), lambda i: (i, 0)),
            pl.BlockSpec((_D, _D), lambda i: (0, 0)),
            pl.BlockSpec((_D, _D), lambda i: (0, 0)),
            pl.BlockSpec((_NC, _RB, 1), lambda i: (0, i, 0)),
        ],
        out_specs=[
            pl.BlockSpec((_RB, _D), lambda i: (i, 0)),
            pl.BlockSpec((_RB, _D), lambda i: (i, 0)),
        ],
        out_shape=[
            jax.ShapeDtypeStruct((_N_PAD, _D), _f32),
            jax.ShapeDtypeStruct((_N_PAD, _D), _f32),
        ],
    )(x_pad, W00, W10, deg_col)


def _tc2_body(s_ref, h0_ref, h1_ref, deg_ref, w0_ref, w1_ref, o0_ref, o1_ref):
    inv0 = lax.rsqrt(deg_ref[0] + 1.0)
    inv1 = lax.rsqrt(deg_ref[1] + 1.0)
    a0 = jnp.maximum((s_ref[0] + h0_ref[...]) * inv0, 0.0)
    a1 = jnp.maximum((s_ref[1] + h1_ref[...]) * inv1, 0.0)
    o0_ref[...] = jnp.dot(a0, w0_ref[...], preferred_element_type=_f32) * inv0
    o1_ref[...] = jnp.dot(a1, w1_ref[...], preferred_element_type=_f32) * inv1


def _tc2(s1, h0, h1, deg_col, W01, W11):
    return pl.pallas_call(
        _tc2_body,
        grid=(_GRID,),
        in_specs=[
            pl.BlockSpec((_NC, _RB, _D), lambda i: (0, i, 0)),
            pl.BlockSpec((_RB, _D), lambda i: (i, 0)),
            pl.BlockSpec((_RB, _D), lambda i: (i, 0)),
            pl.BlockSpec((_NC, _RB, 1), lambda i: (0, i, 0)),
            pl.BlockSpec((_D, _D), lambda i: (0, 0)),
            pl.BlockSpec((_D, _D), lambda i: (0, 0)),
        ],
        out_specs=[
            pl.BlockSpec((_RB, _D), lambda i: (i, 0)),
            pl.BlockSpec((_RB, _D), lambda i: (i, 0)),
        ],
        out_shape=[
            jax.ShapeDtypeStruct((_N_PAD, _D), _f32),
            jax.ShapeDtypeStruct((_N_PAD, _D), _f32),
        ],
    )(s1, h0, h1, deg_col, W01, W11)


def _tc3_body(s_ref, h0_ref, h1_ref, deg_ref, wout_ref, bout_ref, batch_ref,
              out_ref, sums, cnts):
    i = pl.program_id(0)

    @pl.when(i == 0)
    def _():
        sums[...] = jnp.zeros_like(sums)
        cnts[...] = jnp.zeros_like(cnts)

    inv0 = lax.rsqrt(deg_ref[0] + 1.0)
    inv1 = lax.rsqrt(deg_ref[1] + 1.0)
    a0 = (s_ref[0] + h0_ref[...]) * inv0
    a1 = (s_ref[1] + h1_ref[...]) * inv1
    xf = (a0 + a1) * 0.5
    o = jnp.dot(xf, wout_ref[...], preferred_element_type=_f32)
    b2d = batch_ref[0]
    iota_g = lax.broadcasted_iota(_i32, (_NGRAPH, _RB), 0)
    onehot_t = (b2d == iota_g).astype(_f32)
    sums[...] += jnp.dot(onehot_t, o, preferred_element_type=_f32)
    cnts[...] += jnp.sum(onehot_t, axis=1, keepdims=True)

    @pl.when(i == _GRID - 1)
    def _():
        out_ref[...] = sums[...] / jnp.maximum(cnts[...], 1.0) + bout_ref[...]


def _tc3(s2, h0b, h1b, deg_col, W_out_pad, b_out_pad, batch_r):
    return pl.pallas_call(
        _tc3_body,
        grid=(_GRID,),
        in_specs=[
            pl.BlockSpec((_NC, _RB, _D), lambda i: (0, i, 0)),
            pl.BlockSpec((_RB, _D), lambda i: (i, 0)),
            pl.BlockSpec((_RB, _D), lambda i: (i, 0)),
            pl.BlockSpec((_NC, _RB, 1), lambda i: (0, i, 0)),
            pl.BlockSpec((_D, _D), lambda i: (0, 0)),
            pl.BlockSpec((1, _D), lambda i: (0, 0)),
            pl.BlockSpec((1, 1, _RB), lambda i: (i, 0, 0)),
        ],
        out_specs=pl.BlockSpec((_NGRAPH, _D), lambda i: (0, 0)),
        out_shape=jax.ShapeDtypeStruct((_NGRAPH, _D), _f32),
        scratch_shapes=[
            pltpu.VMEM((_NGRAPH, _D), _f32),
            pltpu.VMEM((_NGRAPH, 1), _f32),
        ],
    )(s2, h0b, h1b, deg_col, W_out_pad, b_out_pad, batch_r)


def kernel(x, edge_index_sc, edge_index_fc, batch,
           W0_0, W0_1, W1_0, W1_1, W_out, b_out):
    x_pad = jnp.zeros((_N_PAD, _D), _f32).at[:_N].set(x)

    def prep(ei):
        pad = jnp.full((_E_PAD - _E,), _N, _i32)
        src = jnp.concatenate([ei[0], pad]).reshape(_NS, _NGROUPS, _GROUP, _CHUNK)
        dst = jnp.concatenate([ei[1], pad]).reshape(_NS, _NGROUPS, _GROUP, _CHUNK)
        return src, dst

    src0, dst0 = prep(edge_index_sc)
    src1, dst1 = prep(edge_index_fc)
    src_all = jnp.stack([src0, src1])
    dst_all = jnp.stack([dst0, dst1])

    ones128 = jnp.ones((_CHUNK, _DEGW), _f32)
    zrows = jnp.zeros((_RPT, _D), _f32)

    deg_raw = _sc_deg(dst_all, ones128, zrows)
    deg_col = deg_raw[:, :, 0:1]

    src_all_s = src_all.reshape(_NC, _NS, _SNGROUPS, _SGROUP, _SCHUNK)
    dst_all_s = dst_all.reshape(_NC, _NS, _SNGROUPS, _SGROUP, _SCHUNK)

    h0, h1 = _tc1(x_pad, W0_0, W1_0, deg_col)
    s1 = _sc_scatter(h0, h1, src_all_s, dst_all_s, zrows)
    h0b, h1b = _tc2(s1, h0, h1, deg_col, W0_1, W1_1)
    s2 = _sc_scatter(h0b, h1b, src_all_s, dst_all_s, zrows)

    W_out_pad = jnp.zeros((_D, _D), _f32).at[:, :_NCLASS].set(W_out)
    b_out_pad = jnp.zeros((1, _D), _f32).at[0, :_NCLASS].set(b_out)
    batch_r = jnp.concatenate(
        [batch, jnp.full((_N_PAD - _N,), _NGRAPH, _i32)]).reshape(_GRID, 1, _RB)

    outg = _tc3(s2, h0b, h1b, deg_col, W_out_pad, b_out_pad, batch_r)
    return outg[:, :_NCLASS]
